# baseline (speedup 1.0000x reference)
import functools

import jax
import jax.numpy as jnp
from jax import lax
from jax.experimental import pallas as pl
from jax.experimental.pallas import tpu as pltpu
from jax.experimental.pallas import tpu_sc as plsc

D_MODEL = 128
NUM_CORES = 2
NUM_SUBCORES = 16
NUM_WORKERS = NUM_CORES * NUM_SUBCORES
CHUNK = 128
PER_SLOT = 2
SLOT = CHUNK * PER_SLOT
LANES = 16
NBUF = 3
GROUP_ROWS = 8


def _emb_body(seq, ids_hbm, tti_hbm, tok_tab, tt_tab, pos_tab, out_hbm,
              idx_v, tti_v, cidx_v, tt_v, cb_v, c_sh, rows_v, *sems):
    n_chunks = idx_v.shape[0]
    n_slots = n_chunks // PER_SLOT
    tok_per_w = n_chunks * CHUNK
    s = lax.axis_index("s")
    w = s * NUM_CORES + lax.axis_index("c")
    rbase = w * n_slots
    sem_g = sems[0:NBUF]
    sem_a = sems[NBUF:2 * NBUF]
    sem_o = sems[2 * NBUF:3 * NBUF]

    def issue_gather(t, b):
        for h in range(PER_SLOT):
            pltpu.async_copy(tok_tab.at[idx_v.at[t * PER_SLOT + h]],
                             rows_v.at[b, pl.ds(h * CHUNK, CHUNK)], sem_g[b])

    pltpu.sync_copy(ids_hbm.at[w], idx_v)
    pltpu.sync_copy(tti_hbm.at[w], tti_v)
    issue_gather(0, 0)

    pltpu.sync_copy(tt_tab, tt_v)
    n_groups = 2 * seq // GROUP_ROWS
    for k in range((n_groups + NUM_SUBCORES - 1) // NUM_SUBCORES):
        g = s + k * NUM_SUBCORES

        @pl.when(g < n_groups)
        def _():
            t_g = g // (seq // GROUP_ROWS)
            poff = (g - t_g * (seq // GROUP_ROWS)) * GROUP_ROWS
            pltpu.sync_copy(pos_tab.at[pl.ds(poff, GROUP_ROWS)], cb_v)
            for j in range(D_MODEL // LANES):
                sl = pl.ds(j * LANES, LANES)
                ttrow = tt_v[t_g, sl]
                for p in range(GROUP_ROWS):
                    cb_v[p, sl] = cb_v[p, sl] + ttrow
            pltpu.sync_copy(cb_v, c_sh.at[pl.ds(g * GROUP_ROWS, GROUP_ROWS)])

    def mk_cidx(c, _):
        for j in range(CHUNK // LANES):
            sl = pl.ds(j * LANES, LANES)
            flat = w * tok_per_w + c * CHUNK + j * LANES + lax.iota(
                jnp.int32, LANES)
            pos = lax.rem(flat, seq)
            cidx_v[c, sl] = tti_v[c, sl] * seq + pos
        return 0

    lax.fori_loop(0, n_chunks, mk_cidx, 0)
    plsc.subcore_barrier()

    def wait_bytes(sem, b):
        pltpu.make_async_copy(rows_v.at[b], out_hbm.at[pl.ds(0, SLOT)],
                              sem).wait()

    def group(g, _):
        for u in range(NBUF):
            t = g * NBUF + u
            bG = (u + 1) % NBUF

            @pl.when(t + 1 <= n_slots - 1)
            def _():
                @pl.when(t + 1 >= NBUF)
                def _():
                    wait_bytes(sem_o[bG], bG)

                issue_gather(t + 1, bG)

            bA = u

            @pl.when(t <= n_slots - 1)
            def _():
                wait_bytes(sem_g[bA], bA)
                for h in range(PER_SLOT):
                    pltpu.async_copy(
                        c_sh.at[cidx_v.at[t * PER_SLOT + h]],
                        rows_v.at[bA, pl.ds(h * CHUNK, CHUNK)],
                        sem_a[bA], add=True)

            bO = (u - 1) % NBUF

            @pl.when(jnp.logical_and(t >= 1, t <= n_slots))
            def _():
                c = t - 1
                wait_bytes(sem_a[bO], bO)
                pltpu.async_copy(rows_v.at[bO],
                                 out_hbm.at[pl.ds((rbase + c) * SLOT, SLOT)],
                                 sem_o[bO])
        return 0

    n_steps = n_slots + 1
    lax.fori_loop(0, (n_steps + NBUF - 1) // NBUF, group, 0)
    for b in range(NBUF):
        wait_bytes(sem_o[b], b)


@jax.jit
def _sc_embed(ids3d, tti3d, token_table, tt_table, pos_table):
    total = ids3d.shape[0] * ids3d.shape[1] * ids3d.shape[2]
    n_chunks = total // (NUM_WORKERS * CHUNK)
    seq = pos_table.shape[0]
    mesh = plsc.VectorSubcoreMesh(core_axis_name="c", subcore_axis_name="s")
    run = pl.kernel(
        functools.partial(_emb_body, seq),
        out_type=jax.ShapeDtypeStruct((total, D_MODEL), jnp.float32),
        mesh=mesh,
        scratch_types=[
            pltpu.VMEM((n_chunks, CHUNK), jnp.int32),
            pltpu.VMEM((n_chunks, CHUNK), jnp.int32),
            pltpu.VMEM((n_chunks, CHUNK), jnp.int32),
            pltpu.VMEM((2, D_MODEL), jnp.float32),
            pltpu.VMEM((GROUP_ROWS, D_MODEL), jnp.float32),
            pltpu.VMEM_SHARED((2 * seq, D_MODEL), jnp.float32),
            pltpu.VMEM((NBUF, SLOT, D_MODEL), jnp.float32),
        ] + [pltpu.SemaphoreType.DMA] * (3 * NBUF),
    )
    return run(ids3d, tti3d, token_table, tt_table, pos_table)


def kernel(input_ids, token_type_ids, token_table, tt_table, pos_table):
    batch, seq = input_ids.shape
    total = batch * seq
    shp = (NUM_WORKERS, total // (NUM_WORKERS * CHUNK), CHUNK)
    ids3d = input_ids.reshape(shp)
    tti3d = token_type_ids.reshape(shp)
    out = _sc_embed(ids3d, tti3d, token_table, tt_table, pos_table)
    return out.reshape(batch, seq, D_MODEL)


# speedup vs baseline: 1.0062x; 1.0062x over previous
import functools

import jax
import jax.numpy as jnp
from jax import lax
from jax.experimental import pallas as pl
from jax.experimental.pallas import tpu as pltpu
from jax.experimental.pallas import tpu_sc as plsc

D_MODEL = 128
NUM_CORES = 2
NUM_SUBCORES = 16
NUM_WORKERS = NUM_CORES * NUM_SUBCORES
CHUNK = 128
PER_SLOT = 1
SLOT = CHUNK * PER_SLOT
HALF = 64
LANES = 16
NBUF = 6
GROUP_ROWS = 8


def _emb_body(seq, ids_hbm, tti_hbm, tok_tab, tt_tab, pos_tab, out_hbm,
              idx_v, tti_v, cidx_v, tt_v, cb_v, c_sh, rows_v, *sems):
    n_chunks = idx_v.shape[0]
    n_slots = n_chunks // PER_SLOT
    tok_per_w = n_chunks * CHUNK
    s = lax.axis_index("s")
    w = s * NUM_CORES + lax.axis_index("c")
    rbase = w * n_slots
    sem_g = sems[0:NBUF]
    sem_a = sems[NBUF:2 * NBUF]
    sem_o = sems[2 * NBUF:3 * NBUF]

    def issue_gather(t, b):
        for h in range(CHUNK // HALF):
            pltpu.async_copy(tok_tab.at[idx_v.at[t, pl.ds(h * HALF, HALF)]],
                             rows_v.at[b, pl.ds(h * HALF, HALF)], sem_g[b])

    pltpu.sync_copy(ids_hbm.at[w], idx_v)
    pltpu.sync_copy(tti_hbm.at[w], tti_v)
    issue_gather(0, 0)
    issue_gather(1, 1)

    pltpu.sync_copy(tt_tab, tt_v)
    n_groups = 2 * seq // GROUP_ROWS
    for k in range((n_groups + NUM_SUBCORES - 1) // NUM_SUBCORES):
        g = s + k * NUM_SUBCORES

        @pl.when(g < n_groups)
        def _():
            t_g = g // (seq // GROUP_ROWS)
            poff = (g - t_g * (seq // GROUP_ROWS)) * GROUP_ROWS
            pltpu.sync_copy(pos_tab.at[pl.ds(poff, GROUP_ROWS)], cb_v)
            for j in range(D_MODEL // LANES):
                sl = pl.ds(j * LANES, LANES)
                ttrow = tt_v[t_g, sl]
                for p in range(GROUP_ROWS):
                    cb_v[p, sl] = cb_v[p, sl] + ttrow
            pltpu.sync_copy(cb_v, c_sh.at[pl.ds(g * GROUP_ROWS, GROUP_ROWS)])

    def mk_cidx(c, _):
        for j in range(CHUNK // LANES):
            sl = pl.ds(j * LANES, LANES)
            flat = w * tok_per_w + c * CHUNK + j * LANES + lax.iota(
                jnp.int32, LANES)
            pos = lax.rem(flat, seq)
            cidx_v[c, sl] = tti_v[c, sl] * seq + pos
        return 0

    lax.fori_loop(0, n_chunks, mk_cidx, 0)
    plsc.subcore_barrier()

    def wait_bytes(sem, b):
        pltpu.make_async_copy(rows_v.at[b], out_hbm.at[pl.ds(0, SLOT)],
                              sem).wait()

    def group(g, _):
        for u in range(NBUF):
            t = g * NBUF + u
            bG = (u + 2) % NBUF

            @pl.when(t + 2 <= n_slots - 1)
            def _():
                @pl.when(t + 2 >= NBUF)
                def _():
                    wait_bytes(sem_o[bG], bG)

                issue_gather(t + 2, bG)

            bA = (u - 1) % NBUF

            @pl.when(jnp.logical_and(t >= 1, t <= n_slots))
            def _():
                c = t - 1
                wait_bytes(sem_g[bA], bA)
                pltpu.async_copy(c_sh.at[cidx_v.at[c]], rows_v.at[bA],
                                 sem_a[bA], add=True)

            bO = (u - 2) % NBUF

            @pl.when(jnp.logical_and(t >= 2, t <= n_slots + 1))
            def _():
                c = t - 2
                wait_bytes(sem_a[bO], bO)
                pltpu.async_copy(rows_v.at[bO],
                                 out_hbm.at[pl.ds((rbase + c) * SLOT, SLOT)],
                                 sem_o[bO])
        return 0

    n_steps = n_slots + 2
    lax.fori_loop(0, (n_steps + NBUF - 1) // NBUF, group, 0)
    for b in range(NBUF):
        wait_bytes(sem_o[b], b)


@jax.jit
def _sc_embed(ids3d, tti3d, token_table, tt_table, pos_table):
    total = ids3d.shape[0] * ids3d.shape[1] * ids3d.shape[2]
    n_chunks = total // (NUM_WORKERS * CHUNK)
    seq = pos_table.shape[0]
    mesh = plsc.VectorSubcoreMesh(core_axis_name="c", subcore_axis_name="s")
    run = pl.kernel(
        functools.partial(_emb_body, seq),
        out_type=jax.ShapeDtypeStruct((total, D_MODEL), jnp.float32),
        mesh=mesh,
        scratch_types=[
            pltpu.VMEM((n_chunks, CHUNK), jnp.int32),
            pltpu.VMEM((n_chunks, CHUNK), jnp.int32),
            pltpu.VMEM((n_chunks, CHUNK), jnp.int32),
            pltpu.VMEM((2, D_MODEL), jnp.float32),
            pltpu.VMEM((GROUP_ROWS, D_MODEL), jnp.float32),
            pltpu.VMEM_SHARED((2 * seq, D_MODEL), jnp.float32),
            pltpu.VMEM((NBUF, SLOT, D_MODEL), jnp.float32),
        ] + [pltpu.SemaphoreType.DMA] * (3 * NBUF),
    )
    return run(ids3d, tti3d, token_table, tt_table, pos_table)


def kernel(input_ids, token_type_ids, token_table, tt_table, pos_table):
    batch, seq = input_ids.shape
    total = batch * seq
    shp = (NUM_WORKERS, total // (NUM_WORKERS * CHUNK), CHUNK)
    ids3d = input_ids.reshape(shp)
    tti3d = token_type_ids.reshape(shp)
    out = _sc_embed(ids3d, tti3d, token_table, tt_table, pos_table)
    return out.reshape(batch, seq, D_MODEL)


# speedup vs baseline: 1.0276x; 1.0213x over previous
import functools

import jax
import jax.numpy as jnp
from jax import lax
from jax.experimental import pallas as pl
from jax.experimental.pallas import tpu as pltpu
from jax.experimental.pallas import tpu_sc as plsc

D_MODEL = 128
NUM_CORES = 2
NUM_SUBCORES = 16
NUM_WORKERS = NUM_CORES * NUM_SUBCORES
CHUNK = 128
PER_SLOT = 1
SLOT = CHUNK * PER_SLOT
HALF = 64
LANES = 16
NBUF = 6
GROUP_ROWS = 8


def _emb_body(seq, ids_hbm, tti_hbm, tok_tab, tt_tab, pos_tab, out_hbm,
              idx_v, tti_v, cidx_v, tt_v, cb_v, c_sh, rows_v, *sems):
    n_chunks = idx_v.shape[0]
    n_slots = n_chunks // PER_SLOT
    tok_per_w = n_chunks * CHUNK
    s = lax.axis_index("s")
    w = s * NUM_CORES + lax.axis_index("c")
    rbase = w * n_slots
    sem_g = sems[0:NBUF]
    sem_a = sems[NBUF:2 * NBUF]
    sem_o = sems[2 * NBUF:3 * NBUF]

    def issue_gather(t, b):
        for h in range(CHUNK // HALF):
            pltpu.async_copy(tok_tab.at[idx_v.at[t, pl.ds(h * HALF, HALF)]],
                             rows_v.at[b, pl.ds(h * HALF, HALF)], sem_g[b])

    pltpu.sync_copy(ids_hbm.at[w], idx_v)
    pltpu.sync_copy(tti_hbm.at[w], tti_v)
    issue_gather(0, 0)
    issue_gather(1, 1)

    pltpu.sync_copy(tt_tab, tt_v)
    n_groups = 2 * seq // GROUP_ROWS
    for k in range((n_groups + NUM_SUBCORES - 1) // NUM_SUBCORES):
        g = s + k * NUM_SUBCORES

        @pl.when(g < n_groups)
        def _():
            t_g = g // (seq // GROUP_ROWS)
            poff = (g - t_g * (seq // GROUP_ROWS)) * GROUP_ROWS
            pltpu.sync_copy(pos_tab.at[pl.ds(poff, GROUP_ROWS)], cb_v)
            for j in range(D_MODEL // LANES):
                sl = pl.ds(j * LANES, LANES)
                ttrow = tt_v[t_g, sl]
                for p in range(GROUP_ROWS):
                    cb_v[p, sl] = cb_v[p, sl] + ttrow
            pltpu.sync_copy(cb_v, c_sh.at[pl.ds(g * GROUP_ROWS, GROUP_ROWS)])

    def mk_cidx_row(c):
        for j in range(CHUNK // LANES):
            sl = pl.ds(j * LANES, LANES)
            flat = w * tok_per_w + c * CHUNK + j * LANES + lax.iota(
                jnp.int32, LANES)
            pos = lax.rem(flat, seq)
            cidx_v[c, sl] = tti_v[c, sl] * seq + pos

    mk_cidx_row(0)
    plsc.subcore_barrier()

    def wait_bytes(sem, b):
        pltpu.make_async_copy(rows_v.at[b], out_hbm.at[pl.ds(0, SLOT)],
                              sem).wait()

    def group(g, _):
        for u in range(NBUF):
            t = g * NBUF + u
            bG = (u + 2) % NBUF

            @pl.when(t + 2 <= n_slots - 1)
            def _():
                @pl.when(t + 2 >= NBUF)
                def _():
                    wait_bytes(sem_o[bG], bG)

                issue_gather(t + 2, bG)

            @pl.when(t + 1 <= n_slots - 1)
            def _():
                mk_cidx_row(t + 1)

            bA = (u - 1) % NBUF

            @pl.when(jnp.logical_and(t >= 1, t <= n_slots))
            def _():
                c = t - 1
                wait_bytes(sem_g[bA], bA)
                pltpu.async_copy(c_sh.at[cidx_v.at[c]], rows_v.at[bA],
                                 sem_a[bA], add=True)

            bO = (u - 2) % NBUF

            @pl.when(jnp.logical_and(t >= 2, t <= n_slots + 1))
            def _():
                c = t - 2
                wait_bytes(sem_a[bO], bO)
                pltpu.async_copy(rows_v.at[bO],
                                 out_hbm.at[pl.ds((rbase + c) * SLOT, SLOT)],
                                 sem_o[bO])
        return 0

    n_steps = n_slots + 2
    lax.fori_loop(0, (n_steps + NBUF - 1) // NBUF, group, 0)
    for b in range(NBUF):
        wait_bytes(sem_o[b], b)


@jax.jit
def _sc_embed(ids3d, tti3d, token_table, tt_table, pos_table):
    total = ids3d.shape[0] * ids3d.shape[1] * ids3d.shape[2]
    n_chunks = total // (NUM_WORKERS * CHUNK)
    seq = pos_table.shape[0]
    mesh = plsc.VectorSubcoreMesh(core_axis_name="c", subcore_axis_name="s")
    run = pl.kernel(
        functools.partial(_emb_body, seq),
        out_type=jax.ShapeDtypeStruct((total, D_MODEL), jnp.float32),
        mesh=mesh,
        scratch_types=[
            pltpu.VMEM((n_chunks, CHUNK), jnp.int32),
            pltpu.VMEM((n_chunks, CHUNK), jnp.int32),
            pltpu.VMEM((n_chunks, CHUNK), jnp.int32),
            pltpu.VMEM((2, D_MODEL), jnp.float32),
            pltpu.VMEM((GROUP_ROWS, D_MODEL), jnp.float32),
            pltpu.VMEM_SHARED((2 * seq, D_MODEL), jnp.float32),
            pltpu.VMEM((NBUF, SLOT, D_MODEL), jnp.float32),
        ] + [pltpu.SemaphoreType.DMA] * (3 * NBUF),
    )
    return run(ids3d, tti3d, token_table, tt_table, pos_table)


def kernel(input_ids, token_type_ids, token_table, tt_table, pos_table):
    batch, seq = input_ids.shape
    total = batch * seq
    shp = (NUM_WORKERS, total // (NUM_WORKERS * CHUNK), CHUNK)
    ids3d = input_ids.reshape(shp)
    tti3d = token_type_ids.reshape(shp)
    out = _sc_embed(ids3d, tti3d, token_table, tt_table, pos_table)
    return out.reshape(batch, seq, D_MODEL)


# speedup vs baseline: 1.0327x; 1.0050x over previous
import functools

import jax
import jax.numpy as jnp
from jax import lax
from jax.experimental import pallas as pl
from jax.experimental.pallas import tpu as pltpu
from jax.experimental.pallas import tpu_sc as plsc

D_MODEL = 128
NUM_CORES = 2
NUM_SUBCORES = 16
NUM_WORKERS = NUM_CORES * NUM_SUBCORES
CHUNK = 128
PER_SLOT = 1
SLOT = CHUNK * PER_SLOT
HALF = 64
LANES = 16
NBUF = 6
LEAD = 3
GROUP_ROWS = 8


def _emb_body(seq, ids_hbm, tti_hbm, tok_tab, tt_tab, pos_tab, out_hbm,
              idx_v, tti_v, tt_v, cb_v, c_sh, rows_v, *sems):
    cidx_v = tti_v
    n_chunks = idx_v.shape[0]
    n_slots = n_chunks // PER_SLOT
    tok_per_w = n_chunks * CHUNK
    s = lax.axis_index("s")
    w = s * NUM_CORES + lax.axis_index("c")
    rbase = w * n_slots
    sem_g = sems[0:NBUF]
    sem_a = sems[NBUF:2 * NBUF]
    sem_o = sems[2 * NBUF:3 * NBUF]

    def issue_gather(t, b):
        for h in range(CHUNK // HALF):
            pltpu.async_copy(tok_tab.at[idx_v.at[t, pl.ds(h * HALF, HALF)]],
                             rows_v.at[b, pl.ds(h * HALF, HALF)], sem_g[b])

    pltpu.sync_copy(ids_hbm.at[w], idx_v)
    pltpu.sync_copy(tti_hbm.at[w], tti_v)
    for c0 in range(LEAD):
        issue_gather(c0, c0)

    pltpu.sync_copy(tt_tab, tt_v)
    n_groups = 2 * seq // GROUP_ROWS
    for k in range((n_groups + NUM_SUBCORES - 1) // NUM_SUBCORES):
        g = s + k * NUM_SUBCORES

        @pl.when(g < n_groups)
        def _():
            t_g = g // (seq // GROUP_ROWS)
            poff = (g - t_g * (seq // GROUP_ROWS)) * GROUP_ROWS
            pltpu.sync_copy(pos_tab.at[pl.ds(poff, GROUP_ROWS)], cb_v)
            for j in range(D_MODEL // LANES):
                sl = pl.ds(j * LANES, LANES)
                ttrow = tt_v[t_g, sl]
                for p in range(GROUP_ROWS):
                    cb_v[p, sl] = cb_v[p, sl] + ttrow
            pltpu.sync_copy(cb_v, c_sh.at[pl.ds(g * GROUP_ROWS, GROUP_ROWS)])

    def mk_cidx_row(c):
        for j in range(CHUNK // LANES):
            sl = pl.ds(j * LANES, LANES)
            flat = w * tok_per_w + c * CHUNK + j * LANES + lax.iota(
                jnp.int32, LANES)
            pos = lax.rem(flat, seq)
            cidx_v[c, sl] = tti_v[c, sl] * seq + pos

    mk_cidx_row(0)
    plsc.subcore_barrier()

    def wait_bytes(sem, b):
        pltpu.make_async_copy(rows_v.at[b], out_hbm.at[pl.ds(0, SLOT)],
                              sem).wait()

    def group(g, _):
        for u in range(NBUF):
            t = g * NBUF + u
            bG = (u + LEAD) % NBUF

            @pl.when(t + LEAD <= n_slots - 1)
            def _():
                @pl.when(t + LEAD >= NBUF)
                def _():
                    wait_bytes(sem_o[bG], bG)

                issue_gather(t + LEAD, bG)

            @pl.when(t + 1 <= n_slots - 1)
            def _():
                mk_cidx_row(t + 1)

            bA = (u - 1) % NBUF

            @pl.when(jnp.logical_and(t >= 1, t <= n_slots))
            def _():
                c = t - 1
                wait_bytes(sem_g[bA], bA)
                pltpu.async_copy(c_sh.at[cidx_v.at[c]], rows_v.at[bA],
                                 sem_a[bA], add=True)

            bO = (u - 2) % NBUF

            @pl.when(jnp.logical_and(t >= 2, t <= n_slots + 1))
            def _():
                c = t - 2
                wait_bytes(sem_a[bO], bO)
                pltpu.async_copy(rows_v.at[bO],
                                 out_hbm.at[pl.ds((rbase + c) * SLOT, SLOT)],
                                 sem_o[bO])
        return 0

    n_steps = n_slots + 2
    lax.fori_loop(0, (n_steps + NBUF - 1) // NBUF, group, 0)
    for b in range(NBUF):
        wait_bytes(sem_o[b], b)


@jax.jit
def _sc_embed(ids3d, tti3d, token_table, tt_table, pos_table):
    total = ids3d.shape[0] * ids3d.shape[1] * ids3d.shape[2]
    n_chunks = total // (NUM_WORKERS * CHUNK)
    seq = pos_table.shape[0]
    mesh = plsc.VectorSubcoreMesh(core_axis_name="c", subcore_axis_name="s")
    run = pl.kernel(
        functools.partial(_emb_body, seq),
        out_type=jax.ShapeDtypeStruct((total, D_MODEL), jnp.float32),
        mesh=mesh,
        scratch_types=[
            pltpu.VMEM((n_chunks, CHUNK), jnp.int32),
            pltpu.VMEM((n_chunks, CHUNK), jnp.int32),
            pltpu.VMEM((2, D_MODEL), jnp.float32),
            pltpu.VMEM((GROUP_ROWS, D_MODEL), jnp.float32),
            pltpu.VMEM_SHARED((2 * seq, D_MODEL), jnp.float32),
            pltpu.VMEM((NBUF, SLOT, D_MODEL), jnp.float32),
        ] + [pltpu.SemaphoreType.DMA] * (3 * NBUF),
    )
    return run(ids3d, tti3d, token_table, tt_table, pos_table)


def kernel(input_ids, token_type_ids, token_table, tt_table, pos_table):
    batch, seq = input_ids.shape
    total = batch * seq
    shp = (NUM_WORKERS, total // (NUM_WORKERS * CHUNK), CHUNK)
    ids3d = input_ids.reshape(shp)
    tti3d = token_type_ids.reshape(shp)
    out = _sc_embed(ids3d, tti3d, token_table, tt_table, pos_table)
    return out.reshape(batch, seq, D_MODEL)
